# Initial kernel scaffold; baseline (speedup 1.0000x reference)
import jax
import jax.numpy as jnp
from jax.experimental import pallas as pl


def kernel(adj_rows, adj_cols, adj_vals, image_adj_rows, image_adj_cols, image_adj_vals, text_adj_rows, text_adj_cols, text_adj_vals, text_embedding, uEmbeds, iEmbeds, text_trans):
    raise NotImplementedError("write your pallas kernel here")


# speedup vs baseline: 4.8452x; 4.8452x over previous
import jax
import jax.numpy as jnp
from jax import lax
from jax.experimental import pallas as pl
from jax.experimental.pallas import tpu as pltpu
from jax.experimental.pallas import tpu_sc as plsc

USER = 25000
ITEM = 25000
NN = USER + ITEM
LD = 64
QW = 16
NQ = LD // QW
TD = 384
E = 800000
NNP = 50176

NC = 2
NS = 16
NPASS = NQ // NC
CHUNK = 1024
NG = CHUNK // 128
NCH = 49
EPT = NCH * CHUNK
EP = EPT * NS
ROWS_PER_TILE = NNP // NS
IB = 784


def _spmm_body(init_sum, scale, refs):
    if init_sum:
        (cols_hbm, rows_hbm, vals_hbm, x_hbm, a_hbm, b_hbm, out_hbm,
         colb, rowb, valb, gath, iba, ibb, acc, sem) = refs
    else:
        (cols_hbm, rows_hbm, vals_hbm, x_hbm, out_hbm,
         colb, rowb, valb, gath, iba, ibb, acc, sem) = refs

    c = lax.axis_index("c")
    s = lax.axis_index("s")
    row0 = pl.multiple_of(s * ROWS_PER_TILE, 8)
    egrp0 = s * (EPT // 128)

    if not init_sum:
        def zero_row(r, _):
            iba[r, pl.ds(0, QW)] = jnp.zeros((QW,), jnp.float32)
            return 0

        lax.fori_loop(0, IB, zero_row, 0)

    for q in range(NPASS):
        qi = c * NPASS + q

        if init_sum:
            for t in range(ROWS_PER_TILE // IB):
                r0 = pl.multiple_of(row0 + t * IB, 8)
                pltpu.sync_copy(a_hbm.at[qi].at[pl.ds(r0, IB)], iba)
                pltpu.sync_copy(b_hbm.at[qi].at[pl.ds(r0, IB)], ibb)

                def add_row(r, _):
                    iba[r, pl.ds(0, QW)] = (iba[r, pl.ds(0, QW)]
                                            + ibb[r, pl.ds(0, QW)])
                    return 0

                lax.fori_loop(0, IB, add_row, 0)
                pltpu.sync_copy(iba, acc.at[pl.ds(r0, IB)])
        else:
            for t in range(ROWS_PER_TILE // IB):
                pltpu.sync_copy(iba, acc.at[pl.ds(row0 + t * IB, IB)])

        plsc.subcore_barrier()

        def do_chunk(k, _):
            g0 = pl.multiple_of(egrp0 + k * NG, NG)
            pltpu.sync_copy(cols_hbm.at[pl.ds(g0, NG)], colb)
            pltpu.sync_copy(rows_hbm.at[pl.ds(g0, NG)], rowb)
            pltpu.sync_copy(vals_hbm.at[pl.ds(g0 * 128, CHUNK)], valb)

            descs = [
                pltpu.async_copy(x_hbm.at[qi].at[colb.at[g]],
                                 gath.at[pl.ds(g * 128, 128)], sem)
                for g in range(NG)
            ]
            for d in descs:
                d.wait()

            def scale_grp(jj, _):
                j0 = jj * 16
                v16 = valb[pl.ds(j0, 16)] * scale
                for l in range(16):
                    vv = lax.broadcast(v16[l], (QW,))
                    j = j0 + l
                    gath[j, pl.ds(0, QW)] = gath[j, pl.ds(0, QW)] * vv
                return 0

            lax.fori_loop(0, CHUNK // 16, scale_grp, 0)

            for g in range(NG):
                pltpu.sync_copy(gath.at[pl.ds(g * 128, 128)],
                                acc.at[rowb.at[g]], add=True)
            return 0

        lax.fori_loop(0, NCH, do_chunk, 0)

        plsc.subcore_barrier()

        pltpu.sync_copy(acc.at[pl.ds(row0, ROWS_PER_TILE)],
                        out_hbm.at[qi].at[pl.ds(row0, ROWS_PER_TILE)])


def _make_spmm(init_sum, scale):
    mesh = plsc.VectorSubcoreMesh(core_axis_name="c", subcore_axis_name="s")
    scratch = [
        pltpu.VMEM((NG, 128), jnp.int32),
        pltpu.VMEM((NG, 128), jnp.int32),
        pltpu.VMEM((CHUNK,), jnp.float32),
        pltpu.VMEM((CHUNK, QW), jnp.float32),
        pltpu.VMEM((IB, QW), jnp.float32),
        pltpu.VMEM((IB, QW), jnp.float32),
        pltpu.VMEM_SHARED((NNP, QW), jnp.float32),
        pltpu.SemaphoreType.DMA,
    ]

    def body(*refs):
        _spmm_body(init_sum, scale, refs)

    return pl.kernel(
        body,
        out_type=jax.ShapeDtypeStruct((NQ, NNP, QW), jnp.float32),
        mesh=mesh,
        scratch_types=scratch,
        compiler_params=pltpu.CompilerParams(use_tc_tiling_on_sc=False),
        name="sc_spmm_sum" if init_sum else "sc_spmm",
    )


_spmm_zero = _make_spmm(False, 1.0)
_spmm_zero_half = _make_spmm(False, 0.5)
_spmm_sum = _make_spmm(True, 1.0)


def _text_tc_kernel(x_ref, w_ref, o_ref):
    y = jnp.dot(x_ref[...], w_ref[...], preferred_element_type=jnp.float32)
    y = jnp.where(y >= 0, y, 0.2 * y)
    n = jnp.sqrt(jnp.sum(y * y, axis=1, keepdims=True))
    o_ref[...] = y / jnp.maximum(n, 1e-12)


def _text_feats(text_embedding, text_trans):
    mt = 1000
    return pl.pallas_call(
        _text_tc_kernel,
        grid=(ITEM // mt,),
        in_specs=[
            pl.BlockSpec((mt, TD), lambda i: (i, 0)),
            pl.BlockSpec((TD, LD), lambda i: (0, 0)),
        ],
        out_specs=pl.BlockSpec((mt, LD), lambda i: (i, 0)),
        out_shape=jax.ShapeDtypeStruct((ITEM, LD), jnp.float32),
    )(text_embedding, text_trans)


def _prep_edges(rows, cols, vals):
    pad = EP - E
    r = jnp.pad(rows.astype(jnp.int32), (0, pad)).reshape(EP // 128, 128)
    c = jnp.pad(cols.astype(jnp.int32), (0, pad)).reshape(EP // 128, 128)
    v = jnp.pad(vals.astype(jnp.float32), (0, pad))
    return r, c, v


def _split_stack(m, pad_to=None):
    out = m.reshape(m.shape[0], NQ, QW).transpose(1, 0, 2)
    if pad_to is not None:
        out = jnp.pad(out, ((0, 0), (0, pad_to - out.shape[1]), (0, 0)))
    return out


def kernel(adj_rows, adj_cols, adj_vals,
           image_adj_rows, image_adj_cols, image_adj_vals,
           text_adj_rows, text_adj_cols, text_adj_vals,
           text_embedding, uEmbeds, iEmbeds, text_trans):
    del image_adj_rows, image_adj_cols, image_adj_vals

    aR, aC, aV = _prep_edges(adj_rows, adj_cols, adj_vals)
    tR, tC, tV = _prep_edges(text_adj_rows, text_adj_cols, text_adj_vals)

    u = uEmbeds.astype(jnp.float32)
    i = iEmbeds.astype(jnp.float32)
    iS = _split_stack(i)
    padN = jnp.zeros((NQ, NNP - NN, QW), jnp.float32)
    X1 = _split_stack(jnp.concatenate([u, i], axis=0), NNP)

    tf = _text_feats(text_embedding.astype(jnp.float32),
                     text_trans.astype(jnp.float32))
    X2 = _split_stack(jnp.concatenate([u, tf], axis=0), NNP)

    T = _spmm_zero_half(tC, tR, tV, X1)
    P = _spmm_zero(aC, aR, aV, X2)
    X3 = jnp.concatenate([P[:, :USER], iS, padN], axis=1)
    e0 = _spmm_sum(aC, aR, aV, X3, P, T)
    L1 = _spmm_zero(aC, aR, aV, e0)
    out = _spmm_sum(aC, aR, aV, L1, e0, L1)

    full = out[:, :NN].transpose(1, 0, 2).reshape(NN, LD)
    return full[:USER], full[USER:]
